# baseline (speedup 1.0000x reference)
import functools

import jax
import jax.numpy as jnp
from jax import lax
from jax.experimental import pallas as pl
from jax.experimental.pallas import tpu as pltpu
from jax.experimental.pallas import tpu_sc as plsc

_DIM = 64
_NUM_WORKERS = 32
_CHUNK = 128


def _build_body(n_chunks):
  def body(x_hbm, emb_hbm, out_hbm, idx_v, buf, sem):
    c = lax.axis_index("c")
    s = lax.axis_index("s")
    wid = s * 2 + c
    pltpu.sync_copy(x_hbm.at[wid], idx_v)

    def step(j, _):
      pltpu.async_copy(emb_hbm.at[idx_v.at[j]], buf, sem).wait()
      pltpu.sync_copy(buf, out_hbm.at[wid, j])
      return 0

    lax.fori_loop(0, n_chunks, step, 0)

  return body


@functools.partial(jax.jit, static_argnums=())
def _embed(x_flat, embedding):
  n_total = x_flat.shape[0]
  per_w = n_total // _NUM_WORKERS
  n_chunks = per_w // _CHUNK
  x3 = x_flat.reshape(_NUM_WORKERS, n_chunks, _CHUNK)

  mesh = plsc.VectorSubcoreMesh(core_axis_name="c", subcore_axis_name="s")
  kfn = pl.kernel(
      _build_body(n_chunks),
      out_type=jax.ShapeDtypeStruct(
          (_NUM_WORKERS, n_chunks, _CHUNK, _DIM), jnp.float32),
      mesh=mesh,
      scratch_types=[
          pltpu.VMEM((n_chunks, _CHUNK), jnp.int32),
          pltpu.VMEM((_CHUNK, _DIM), jnp.float32),
          pltpu.SemaphoreType.DMA,
      ],
      compiler_params=pltpu.CompilerParams(use_tc_tiling_on_sc=False),
  )
  return kfn(x3, embedding)


def kernel(x, embedding):
  b, t = x.shape
  out = _embed(x.reshape(-1), embedding)
  return out.reshape(b, t, _DIM)


# speedup vs baseline: 1.1167x; 1.1167x over previous
import functools

import jax
import jax.numpy as jnp
from jax import lax
from jax.experimental import pallas as pl
from jax.experimental.pallas import tpu as pltpu
from jax.experimental.pallas import tpu_sc as plsc

_DIM = 64
_NUM_WORKERS = 32
_CHUNK = 128
_NBUF = 8
_H = 4


def _build_body(n_chunks):
  assert n_chunks % _NBUF == 0 and n_chunks >= 3 * _NBUF

  def body(x_hbm, emb_hbm, out_hbm, idx_v, bufs, gsems, ssems):
    c = lax.axis_index("c")
    s = lax.axis_index("s")
    wid = s * 2 + c
    pltpu.sync_copy(x_hbm.at[wid], idx_v)

    def start_gather(j, b):
      pltpu.make_async_copy(
          emb_hbm.at[idx_v.at[j]], bufs[b], gsems[b]).start()

    def wait_gather(b):
      pltpu.make_async_copy(
          emb_hbm.at[idx_v.at[0]], bufs[b], gsems[b]).wait()

    def start_store(j, b):
      pltpu.make_async_copy(bufs[b], out_hbm.at[wid, j], ssems[b]).start()

    def wait_store(b):
      pltpu.make_async_copy(bufs[b], out_hbm.at[wid, 0], ssems[b]).wait()

    for j in range(_H):
      start_gather(j, j % _NBUF)

    for b in range(_NBUF):
      jg = b + _H
      if jg >= _NBUF:
        wait_store(jg % _NBUF)
      start_gather(jg, jg % _NBUF)
      wait_gather(b)
      start_store(b, b)

    def outer(k, _):
      j0 = k * _NBUF
      for b in range(_NBUF):
        j = j0 + b
        bg = (b + _H) % _NBUF
        wait_store(bg)
        start_gather(j + _H, bg)
        wait_gather(b)
        start_store(j, b)
      return 0

    lax.fori_loop(1, n_chunks // _NBUF - 1, outer, 0)

    for b in range(_NBUF):
      j = n_chunks - _NBUF + b
      jg = j + _H
      if jg < n_chunks:
        bg = jg % _NBUF
        wait_store(bg)
        start_gather(jg, bg)
      wait_gather(b)
      start_store(j, b)

    for b in range(_NBUF):
      wait_store(b)

  return body


@functools.partial(jax.jit, static_argnums=())
def _embed(x_flat, embedding):
  n_total = x_flat.shape[0]
  per_w = n_total // _NUM_WORKERS
  n_chunks = per_w // _CHUNK
  x3 = x_flat.reshape(_NUM_WORKERS, n_chunks, _CHUNK)

  mesh = plsc.VectorSubcoreMesh(core_axis_name="c", subcore_axis_name="s")
  kfn = pl.kernel(
      _build_body(n_chunks),
      out_type=jax.ShapeDtypeStruct(
          (_NUM_WORKERS, n_chunks, _CHUNK, _DIM), jnp.float32),
      mesh=mesh,
      scratch_types=[
          pltpu.VMEM((n_chunks, _CHUNK), jnp.int32),
          [pltpu.VMEM((_CHUNK, _DIM), jnp.float32) for _ in range(_NBUF)],
          [pltpu.SemaphoreType.DMA for _ in range(_NBUF)],
          [pltpu.SemaphoreType.DMA for _ in range(_NBUF)],
      ],
      compiler_params=pltpu.CompilerParams(use_tc_tiling_on_sc=False),
  )
  return kfn(x3, embedding)


def kernel(x, embedding):
  b, t = x.shape
  out = _embed(x.reshape(-1), embedding)
  return out.reshape(b, t, _DIM)


# speedup vs baseline: 1.1454x; 1.0257x over previous
import functools

import jax
import jax.numpy as jnp
from jax import lax
from jax.experimental import pallas as pl
from jax.experimental.pallas import tpu as pltpu
from jax.experimental.pallas import tpu_sc as plsc

_DIM = 64
_NUM_WORKERS = 32
_CHUNK = 128
_NS = 8
_H = 4


def _build_body(n_t):
  def body(xt_hbm, emb_hbm, out_hbm, idx_v, bufs, gsems, ssems):
    c = lax.axis_index("c")
    s = lax.axis_index("s")
    wid = s * 2 + c
    pltpu.sync_copy(xt_hbm.at[wid], idx_v)

    def start_gather(t, b):
      pltpu.make_async_copy(
          emb_hbm.at[idx_v.at[t]], bufs[b], gsems[b]).start()

    def wait_gather(b):
      pltpu.make_async_copy(
          emb_hbm.at[idx_v.at[0]], bufs[b], gsems[b]).wait()

    def start_store(t, b):
      pltpu.make_async_copy(bufs[b], out_hbm.at[t, wid], ssems[b]).start()

    def wait_store(b):
      pltpu.make_async_copy(bufs[b], out_hbm.at[0, wid], ssems[b]).wait()

    for t in range(_H):
      start_gather(t, t % _NS)

    for b in range(_NS):
      tg = b + _H
      if tg >= _NS:
        wait_store(tg % _NS)
      start_gather(tg, tg % _NS)
      wait_gather(b)
      start_store(b, b)

    def outer(k, _):
      t0 = k * _NS
      for b in range(_NS):
        t = t0 + b
        bg = (b + _H) % _NS
        wait_store(bg)
        start_gather(t + _H, bg)
        wait_gather(b)
        start_store(t, b)
      return 0

    lax.fori_loop(1, n_t // _NS - 1, outer, 0)

    for b in range(_NS):
      t = n_t - _NS + b
      tg = t + _H
      if tg < n_t:
        bg = tg % _NS
        wait_store(bg)
        start_gather(tg, bg)
      wait_gather(b)
      start_store(t, b)

    for b in range(_NS):
      wait_store(b)

  return body


@functools.partial(jax.jit, static_argnums=())
def _embed(xq, embedding):
  n_w, n_t, _ = xq.shape
  mesh = plsc.VectorSubcoreMesh(core_axis_name="c", subcore_axis_name="s")
  kfn = pl.kernel(
      _build_body(n_t),
      out_type=jax.ShapeDtypeStruct(
          (n_t, _NUM_WORKERS, _CHUNK, _DIM), jnp.float32),
      mesh=mesh,
      scratch_types=[
          pltpu.VMEM((n_t, _CHUNK), jnp.int32),
          [pltpu.VMEM((_CHUNK, _DIM), jnp.float32) for _ in range(_NS)],
          [pltpu.SemaphoreType.DMA for _ in range(_NS)],
          [pltpu.SemaphoreType.DMA for _ in range(_NS)],
      ],
      compiler_params=pltpu.CompilerParams(use_tc_tiling_on_sc=False),
  )
  return kfn(xq, embedding)


def kernel(x, embedding):
  b, t = x.shape
  xq = x.T.reshape(t, _NUM_WORKERS, _CHUNK).transpose(1, 0, 2)
  out = _embed(xq, embedding)
  return out.transpose(1, 2, 0, 3).reshape(b, t, _DIM)
